# Initial kernel scaffold; baseline (speedup 1.0000x reference)
import jax
import jax.numpy as jnp
from jax.experimental import pallas as pl


def kernel(seq1, adj, sparse, W1, b1, W2, b2, w0):
    raise NotImplementedError("write your pallas kernel here")


# speedup vs baseline: 1.8065x; 1.8065x over previous
import jax
import jax.numpy as jnp
from jax.experimental import pallas as pl
from jax.experimental.pallas import tpu as pltpu

_BI = 400


def _fc1_body(x_ref, w1_ref, b1_ref, o_ref):
    o_ref[...] = (
        jnp.dot(x_ref[...], w1_ref[...], preferred_element_type=jnp.float32)
        + b1_ref[...]
    )


def _layer1_body(adj_ref, h_ref, w2_ref, b2_ref, o_ref):
    p = jnp.dot(adj_ref[...], h_ref[...], preferred_element_type=jnp.float32)
    r = jnp.maximum(p, 0.0)
    o_ref[...] = (
        jnp.dot(r, w2_ref[...], preferred_element_type=jnp.float32) + b2_ref[...]
    )


def _layer2_body(adj_ref, h_ref, w0_ref, o_ref):
    p = jnp.dot(adj_ref[...], h_ref[...], preferred_element_type=jnp.float32)
    o_ref[...] = jnp.maximum(p, 0.0) * w0_ref[0, 0]


def kernel(seq1, adj, sparse, W1, b1, W2, b2, w0):
    del sparse
    n = seq1.shape[2]
    d_in = seq1.shape[3]
    d_h = W1.shape[1]
    d_out = W2.shape[1]
    x = seq1.reshape(n, d_in)
    a = adj.reshape(n, n)
    ni = n // _BI

    h1 = pl.pallas_call(
        _fc1_body,
        grid=(ni,),
        in_specs=[
            pl.BlockSpec((_BI, d_in), lambda i: (i, 0)),
            pl.BlockSpec((d_in, d_h), lambda i: (0, 0)),
            pl.BlockSpec((1, d_h), lambda i: (0, 0)),
        ],
        out_specs=pl.BlockSpec((_BI, d_h), lambda i: (i, 0)),
        out_shape=jax.ShapeDtypeStruct((n, d_h), jnp.float32),
        compiler_params=pltpu.CompilerParams(dimension_semantics=("parallel",)),
    )(x, W1, b1.reshape(1, d_h))

    h2 = pl.pallas_call(
        _layer1_body,
        grid=(ni,),
        in_specs=[
            pl.BlockSpec((_BI, n), lambda i: (i, 0)),
            pl.BlockSpec((n, d_h), lambda i: (0, 0)),
            pl.BlockSpec((d_h, d_out), lambda i: (0, 0)),
            pl.BlockSpec((1, d_out), lambda i: (0, 0)),
        ],
        out_specs=pl.BlockSpec((_BI, d_out), lambda i: (i, 0)),
        out_shape=jax.ShapeDtypeStruct((n, d_out), jnp.float32),
        compiler_params=pltpu.CompilerParams(dimension_semantics=("parallel",)),
    )(a, h1, W2, b2.reshape(1, d_out))

    out = pl.pallas_call(
        _layer2_body,
        grid=(ni,),
        in_specs=[
            pl.BlockSpec((_BI, n), lambda i: (i, 0)),
            pl.BlockSpec((n, d_out), lambda i: (0, 0)),
            pl.BlockSpec((1, 1), lambda i: (0, 0)),
        ],
        out_specs=pl.BlockSpec((_BI, d_out), lambda i: (i, 0)),
        out_shape=jax.ShapeDtypeStruct((n, d_out), jnp.float32),
        compiler_params=pltpu.CompilerParams(dimension_semantics=("parallel",)),
    )(a, h2, w0.reshape(1, 1))

    return out.reshape(1, n, d_out)
